# baseline (speedup 1.0000x reference)
import jax
import jax.numpy as jnp
from jax.experimental import pallas as pl
from jax.experimental.pallas import tpu as pltpu

SOM_H, SOM_W, D = 64, 64, 512
HW = SOM_H * SOM_W
BATCH = 4096
TB = 1024
CW = 1024
NC = HW // CW


def _som_kernel(x_ref, w_ref, coord_ref, idx_ref, wsq_ref):
    @pl.when(pl.program_id(0) == 0)
    def _():
        w = w_ref[...]
        wsq_ref[...] = jnp.sum(w * w, axis=1)[None, :]

    x = x_ref[...]
    x_sq = jnp.sum(x * x, axis=1, keepdims=True)

    def chunk_dot(c):
        return jax.lax.dot_general(
            x, w_ref[pl.ds(c * CW, CW), :], (((1,), (1,)), ((), ())),
            preferred_element_type=jnp.float32,
        )

    def chunk_epi(c, cross, run):
        dist = (x_sq - 2.0 * cross) + wsq_ref[:, c * CW:(c + 1) * CW]
        m_c = jnp.min(dist, axis=1, keepdims=True)
        i_c = jnp.argmin(dist, axis=1).astype(jnp.int32)[:, None] + c * CW
        if run is None:
            return m_c, i_c
        m_run, i_run = run
        upd = m_c < m_run
        m_run = jnp.minimum(m_run, m_c)
        i_run = jnp.where(upd, i_c, i_run)
        return m_run, i_run

    run = None
    cross_prev = chunk_dot(0)
    for c in range(1, NC):
        cross_c = chunk_dot(c)
        run = chunk_epi(c - 1, cross_prev, run)
        cross_prev = cross_c
    run = chunk_epi(NC - 1, cross_prev, run)

    idx = run[1][:, 0]
    idx_ref[...] = idx[:, None]
    coord_ref[...] = jnp.stack([idx // SOM_W, idx % SOM_W], axis=1)


def kernel(x, weights):
    wf = weights.reshape(HW, D)
    grid = (BATCH // TB,)
    coords, idx = pl.pallas_call(
        _som_kernel,
        grid=grid,
        in_specs=[
            pl.BlockSpec((TB, D), lambda i: (i, 0)),
            pl.BlockSpec((HW, D), lambda i: (0, 0)),
        ],
        out_specs=[
            pl.BlockSpec((TB, 2), lambda i: (i, 0)),
            pl.BlockSpec((TB, 1), lambda i: (i, 0)),
        ],
        out_shape=[
            jax.ShapeDtypeStruct((BATCH, 2), jnp.int32),
            jax.ShapeDtypeStruct((BATCH, 1), jnp.int32),
        ],
        scratch_shapes=[pltpu.VMEM((1, HW), jnp.float32)],
    )(x, wf)
    return coords, idx[:, 0]


# speedup vs baseline: 1.2476x; 1.2476x over previous
import jax
import jax.numpy as jnp
from jax.experimental import pallas as pl
from jax.experimental.pallas import tpu as pltpu

SOM_H, SOM_W, D = 64, 64, 512
HW = SOM_H * SOM_W
BATCH = 4096
TB = 1024


def _som_kernel(x_ref, w_ref, coord_ref, idx_ref, wsq_ref):
    w = w_ref[...].reshape(HW, D)

    @pl.when(pl.program_id(0) == 0)
    def _():
        wsq_ref[...] = jnp.sum(w * w, axis=1)[None, :]

    x = x_ref[...]
    x_sq = jnp.sum(x * x, axis=1, keepdims=True)
    cross = jax.lax.dot_general(
        x, w, (((1,), (1,)), ((), ())),
        preferred_element_type=jnp.float32,
    )
    dist = (x_sq - 2.0 * cross) + wsq_ref[...]
    idx = jnp.argmin(dist, axis=1).astype(jnp.int32)
    idx_ref[...] = idx[:, None]
    coord_ref[...] = jnp.stack([idx // SOM_W, idx % SOM_W], axis=1)


def kernel(x, weights):
    grid = (BATCH // TB,)
    coords, idx = pl.pallas_call(
        _som_kernel,
        grid=grid,
        in_specs=[
            pl.BlockSpec((TB, D), lambda i: (i, 0)),
            pl.BlockSpec((SOM_H, SOM_W, D), lambda i: (0, 0, 0)),
        ],
        out_specs=[
            pl.BlockSpec((TB, 2), lambda i: (i, 0)),
            pl.BlockSpec((TB, 1), lambda i: (i, 0)),
        ],
        out_shape=[
            jax.ShapeDtypeStruct((BATCH, 2), jnp.int32),
            jax.ShapeDtypeStruct((BATCH, 1), jnp.int32),
        ],
        scratch_shapes=[pltpu.VMEM((1, HW), jnp.float32)],
    )(x, weights)
    return coords, idx[:, 0]
